# Initial kernel scaffold; baseline (speedup 1.0000x reference)
import jax
import jax.numpy as jnp
from jax.experimental import pallas as pl


def kernel(node_feats, edge_index, mask, exp_temp, rand_coef, W_gcn, b_gcn, W_dep1, b_dep1, W_dep2, b_dep2, W_bi, W_bw1, b_bw1, W_bw2, b_bw2):
    raise NotImplementedError("write your pallas kernel here")


# speedup vs baseline: 2.4167x; 2.4167x over previous
import functools

import jax
import jax.numpy as jnp
from jax import lax
from jax.experimental import pallas as pl
from jax.experimental.pallas import tpu as pltpu
from jax.experimental.pallas import tpu_sc as plsc

_B, _N, _D, _E = 32, 128, 1024, 65536
_BN = _B * _N
_Q = _D // 4
_NS = 16
_NC = 2
_CH = 32
_EPT = _E // (_NS * _CH)
_RPT = _BN // _NS


def _sc_body(x0r, x1r, x2r, x3r, srcr, dstr, zr, zdr,
             o0r, o1r, o2r, o3r, degpr,
             src_v, dst_v, rows_v, deg_v, shared, sem):
    c = lax.axis_index("c")
    s = lax.axis_index("s")
    wid = s * _NC + c
    pltpu.sync_copy(srcr.at[pl.ds(s * _EPT, _EPT)], src_v)
    pltpu.sync_copy(dstr.at[pl.ds(s * _EPT, _EPT)], dst_v)
    pltpu.sync_copy(zdr, deg_v)

    ones16 = jnp.ones((16,), jnp.float32)

    def dbody(i, carry):
        d16 = dst_v[i >> 1, pl.ds((i & 1) * 16, 16)]
        plsc.addupdate_scatter(
            deg_v,
            [lax.shift_right_logical(d16, 7), lax.bitwise_and(d16, 127)],
            ones16)
        return carry

    lax.fori_loop(0, _EPT * (_CH // 16), dbody, 0)

    tables = [x0r, x1r, x2r, x3r]
    outs = [o0r, o1r, o2r, o3r]
    for p in range(2):
        pltpu.sync_copy(zr.at[pl.ds(s * _RPT, _RPT)],
                        shared.at[pl.ds(s * _RPT, _RPT)])
        plsc.subcore_barrier()

        def _mk_gather(table):
            def go():
                def cbody(j, carry):
                    pltpu.async_copy(table.at[src_v.at[j]], rows_v, sem).wait()
                    pltpu.sync_copy(rows_v, shared.at[dst_v.at[j]], add=True)
                    return carry
                lax.fori_loop(0, _EPT, cbody, 0)
            return go

        pl.when(c == 0)(_mk_gather(tables[2 * p]))
        pl.when(c == 1)(_mk_gather(tables[2 * p + 1]))
        plsc.subcore_barrier()

        def _mk_out(o):
            def go():
                pltpu.sync_copy(shared.at[pl.ds(s * _RPT, _RPT)],
                                o.at[pl.ds(s * _RPT, _RPT)])
            return go

        pl.when(c == 0)(_mk_out(outs[2 * p]))
        pl.when(c == 1)(_mk_out(outs[2 * p + 1]))
        plsc.subcore_barrier()

    pltpu.sync_copy(deg_v, degpr.at[wid])


def _sc_aggregate(x0, x1, x2, x3, src2, dst2, zrows, zdeg):
    f32 = jnp.float32
    out_type = [jax.ShapeDtypeStruct((_BN, _Q), f32) for _ in range(4)]
    out_type.append(jax.ShapeDtypeStruct((_NC * _NS, _BN // 128, 128), f32))
    kern = pl.kernel(
        _sc_body,
        out_type=out_type,
        mesh=plsc.VectorSubcoreMesh(core_axis_name="c", subcore_axis_name="s"),
        compiler_params=pltpu.CompilerParams(needs_layout_passes=False,
                                              use_tc_tiling_on_sc=False),
        scratch_types=[
            pltpu.VMEM((_EPT, _CH), jnp.int32),
            pltpu.VMEM((_EPT, _CH), jnp.int32),
            pltpu.VMEM((_CH, _Q), f32),
            pltpu.VMEM((_BN // 128, 128), f32),
            pltpu.VMEM_SHARED((_BN, _Q), f32),
            pltpu.SemaphoreType.DMA,
        ],
    )
    return kern(x0, x1, x2, x3, src2, dst2, zrows, zdeg)


def _softmax_mix_log(logits, valid, exp_temp, rand_coef):
    neg = jnp.float32(-1e9)
    masked = jnp.where(valid > 0, logits / exp_temp, neg)
    m = jnp.max(masked, axis=-1, keepdims=True)
    e = jnp.exp(masked - m)
    p = e / jnp.sum(e, axis=-1, keepdims=True)
    uni = valid / jnp.clip(jnp.sum(valid, axis=-1, keepdims=True), 1.0, None)
    probs = (1.0 - rand_coef) * p + rand_coef * uni
    return jnp.log(probs + 1e-20)


def _tc_main_body(xr, ar, dgr, mr, wgr, bgr, w1dr, b1dr, w2dr, w1br, b1br,
                  w2br, sclr, hr, lpdr, lbwr):
    exp_temp = sclr[0, 0]
    rand_coef = sclr[0, 1]
    b_dep2 = sclr[0, 2]
    b_bw2 = sclr[0, 3]
    deg = jnp.clip(0.5 * jnp.sum(dgr[...], axis=0), 1.0, None)
    x = xr[...] + ar[...] / deg[:, None]
    h = jnp.maximum(jnp.dot(x, wgr[...], preferred_element_type=jnp.float32)
                    + bgr[...], 0.0)
    hr[...] = h
    hd = jnp.maximum(jnp.dot(h, w1dr[...], preferred_element_type=jnp.float32)
                     + b1dr[...], 0.0)
    ld = jnp.sum(hd * w2dr[...], axis=1).reshape(4, _N) + b_dep2
    hb = jnp.maximum(jnp.dot(h, w1br[...], preferred_element_type=jnp.float32)
                     + b1br[...], 0.0)
    lb = jnp.sum(hb * w2br[...], axis=1).reshape(4, _N) + b_bw2
    dep_mask = jnp.max(mr[...], axis=1)
    lpdr[...] = _softmax_mix_log(ld, dep_mask, exp_temp, rand_coef)[None]
    col = lax.broadcasted_iota(jnp.int32, (4, _N), 1)
    bmask = (dep_mask <= 0.0) & (col > 0)
    lbwr[...] = jnp.where(bmask, lb, jnp.float32(-1e9))[None]


def _tc_main(x, agg, degp, mask_f, W_gcn, b_gcn, W_dep1, b_dep1, w_dep2,
             W_bw1, b_bw1, w_bw2, scal):
    f32 = jnp.float32
    grid = _BN // 512
    return pl.pallas_call(
        _tc_main_body,
        grid=(grid,),
        in_specs=[
            pl.BlockSpec((512, _D), lambda i: (i, 0)),
            pl.BlockSpec((512, _D), lambda i: (i, 0)),
            pl.BlockSpec((_NC * _NS, 512), lambda i: (0, i)),
            pl.BlockSpec((4, _N, _N), lambda i: (i, 0, 0)),
            pl.BlockSpec((_D, _D), lambda i: (0, 0)),
            pl.BlockSpec((1, _D), lambda i: (0, 0)),
            pl.BlockSpec((_D, 512), lambda i: (0, 0)),
            pl.BlockSpec((1, 512), lambda i: (0, 0)),
            pl.BlockSpec((1, 512), lambda i: (0, 0)),
            pl.BlockSpec((_D, 512), lambda i: (0, 0)),
            pl.BlockSpec((1, 512), lambda i: (0, 0)),
            pl.BlockSpec((1, 512), lambda i: (0, 0)),
            pl.BlockSpec(memory_space=pltpu.SMEM),
        ],
        out_specs=[
            pl.BlockSpec((512, _D), lambda i: (i, 0)),
            pl.BlockSpec((1, 4, _N), lambda i: (i, 0, 0)),
            pl.BlockSpec((1, 4, _N), lambda i: (i, 0, 0)),
        ],
        out_shape=[
            jax.ShapeDtypeStruct((_BN, _D), f32),
            jax.ShapeDtypeStruct((_B // 4, 4, _N), f32),
            jax.ShapeDtypeStruct((_B // 4, 4, _N), f32),
        ],
    )(x, agg, degp, mask_f, W_gcn, b_gcn, W_dep1, b_dep1, w_dep2, W_bw1,
      b_bw1, w_bw2, scal)


def _tc_head_body(hr, mr, wr, ur, vr, lpdr, depr, sclr, lphr, lpfdr):
    b = pl.program_id(0)
    exp_temp = sclr[0, 0]
    rand_coef = sclr[0, 1]
    s_bi = sclr[0, 2]
    d = depr[b]
    hblk = hr[0]
    oh = (lax.broadcasted_iota(jnp.int32, (1, _N), 1) == d).astype(jnp.float32)
    hd = jnp.dot(oh, hblk, preferred_element_type=jnp.float32)
    v = lax.dot_general(hd, wr[...], (((1,), (1,)), ((), ())),
                        preferred_element_type=jnp.float32) + ur[...]
    t = jnp.sum(vr[...] * hd) + s_bi
    lh = lax.dot_general(v, hblk, (((1,), (1,)), ((), ())),
                         preferred_element_type=jnp.float32) + t
    hm = lax.dot_general(mr[0], oh, (((1,), (1,)), ((), ())),
                         preferred_element_type=jnp.float32).reshape(1, _N)
    lphr[...] = _softmax_mix_log(lh, hm, exp_temp, rand_coef)[None]
    lpfdr[...] = jnp.sum(lpdr[0] * oh, axis=-1, keepdims=True)[None]


def _tc_head(h3, mask_f, Wc, u_bi, w_bi, logp_dep, dep_ids, scal):
    f32 = jnp.float32
    return pl.pallas_call(
        _tc_head_body,
        grid=(_B,),
        in_specs=[
            pl.BlockSpec((1, _N, _D), lambda b: (b, 0, 0)),
            pl.BlockSpec((1, _N, _N), lambda b: (b, 0, 0)),
            pl.BlockSpec((_D, _D), lambda b: (0, 0)),
            pl.BlockSpec((1, _D), lambda b: (0, 0)),
            pl.BlockSpec((1, _D), lambda b: (0, 0)),
            pl.BlockSpec((1, 1, _N), lambda b: (b, 0, 0)),
            pl.BlockSpec(memory_space=pltpu.SMEM),
            pl.BlockSpec(memory_space=pltpu.SMEM),
        ],
        out_specs=[
            pl.BlockSpec((1, 1, _N), lambda b: (b, 0, 0)),
            pl.BlockSpec((1, 1, 1), lambda b: (b, 0, 0)),
        ],
        out_shape=[
            jax.ShapeDtypeStruct((_B, 1, _N), f32),
            jax.ShapeDtypeStruct((_B, 1, 1), f32),
        ],
    )(h3, mask_f, Wc, u_bi, w_bi, logp_dep.reshape(_B, 1, _N), dep_ids, scal)


def kernel(node_feats, edge_index, mask, exp_temp, rand_coef, W_gcn, b_gcn,
           W_dep1, b_dep1, W_dep2, b_dep2, W_bi, W_bw1, b_bw1, W_bw2, b_bw2):
    f32 = jnp.float32
    x = node_feats.astype(f32)
    src2 = edge_index[0].reshape(_E // _CH, _CH).astype(jnp.int32)
    dst2 = edge_index[1].reshape(_E // _CH, _CH).astype(jnp.int32)
    mask_f = mask.astype(f32)

    xq = [x[:, i * _Q:(i + 1) * _Q] for i in range(4)]
    zrows = jnp.zeros((_BN, _Q), f32)
    zdeg = jnp.zeros((_BN // 128, 128), f32)
    o0, o1, o2, o3, degp = _sc_aggregate(xq[0], xq[1], xq[2], xq[3], src2,
                                         dst2, zrows, zdeg)
    agg = jnp.concatenate([o0, o1, o2, o3], axis=1)
    degp2 = degp.reshape(_NC * _NS, _BN)

    et = jnp.asarray(exp_temp, f32)
    rc = jnp.asarray(rand_coef, f32)
    scal1 = jnp.stack([et, rc, b_dep2.reshape(()).astype(f32),
                       b_bw2.reshape(()).astype(f32)]).reshape(1, 4)
    h, logp_dep, logits_bw = _tc_main(
        x, agg, degp2, mask_f, W_gcn, b_gcn.reshape(1, _D),
        W_dep1, b_dep1.reshape(1, 512), W_dep2.reshape(1, 512),
        W_bw1, b_bw1.reshape(1, 512), W_bw2.reshape(1, 512), scal1)
    logp_dep = logp_dep.reshape(_B, _N)
    logits_bw = logits_bw.reshape(_B, _N)

    dep_ids = jax.random.categorical(jax.random.key(1), logp_dep,
                                     axis=-1)[:, None]

    Wc = W_bi[:_D, :_D]
    u_bi = W_bi[:_D, _D].reshape(1, _D)
    w_bi = W_bi[_D, :_D].reshape(1, _D)
    scal2 = jnp.stack([et, rc, W_bi[_D, _D].astype(f32)]).reshape(1, 3)
    logp_head, log_pF_dep = _tc_head(
        h.reshape(_B, _N, _D), mask_f, Wc, u_bi, w_bi, logp_dep,
        dep_ids[:, 0].astype(jnp.int32), scal2)
    logp_head = logp_head.reshape(_B, _N)
    log_pF_dep = log_pF_dep.reshape(_B, 1)

    head_ids = jax.random.categorical(jax.random.key(2), logp_head,
                                      axis=-1)[:, None]
    log_pF_head = jnp.take_along_axis(logp_head, head_ids, axis=-1)
    return (head_ids, dep_ids), (log_pF_head, log_pF_dep), logits_bw


# speedup vs baseline: 3.4587x; 1.4312x over previous
import functools

import jax
import jax.numpy as jnp
from jax import lax
from jax.experimental import pallas as pl
from jax.experimental.pallas import tpu as pltpu
from jax.experimental.pallas import tpu_sc as plsc

_B, _N, _D, _E = 32, 128, 1024, 65536
_BN = _B * _N
_Q = _D // 4
_NS = 16
_NC = 2
_CH = 32
_EPT = _E // (_NS * _CH)
_RPT = _BN // _NS


def _sc_body(x0r, x1r, x2r, x3r, srcr, dstr, zr, zdr,
             o0r, o1r, o2r, o3r, degpr,
             src_v, dst_v, rows_v, rows_w, deg_v, shared, sem, sem2):
    c = lax.axis_index("c")
    s = lax.axis_index("s")
    wid = s * _NC + c
    pltpu.sync_copy(srcr.at[pl.ds(s * _EPT, _EPT)], src_v)
    pltpu.sync_copy(dstr.at[pl.ds(s * _EPT, _EPT)], dst_v)
    pltpu.sync_copy(zdr, deg_v)

    ones16 = jnp.ones((16,), jnp.float32)

    def dbody(i, carry):
        d16 = dst_v[i >> 1, pl.ds((i & 1) * 16, 16)]
        plsc.addupdate_scatter(
            deg_v,
            [lax.shift_right_logical(d16, 7), lax.bitwise_and(d16, 127)],
            ones16)
        return carry

    lax.fori_loop(0, _EPT * (_CH // 16), dbody, 0)

    tables = [x0r, x1r, x2r, x3r]
    outs = [o0r, o1r, o2r, o3r]
    for p in range(2):
        pltpu.sync_copy(zr.at[pl.ds(s * _RPT, _RPT)],
                        shared.at[pl.ds(s * _RPT, _RPT)])
        plsc.subcore_barrier()

        def _mk_gather(table):
            def go():
                bufs = (rows_v, rows_w)
                sems = (sem, sem2)
                for b in range(2):
                    pltpu.async_copy(table.at[src_v.at[b]], bufs[b], sems[b])

                def cbody(i, carry):
                    j0 = i * 2
                    for b in range(2):
                        j = j0 + b
                        pltpu.make_async_copy(
                            table.at[src_v.at[j]], bufs[b], sems[b]).wait()
                        pltpu.sync_copy(bufs[b], shared.at[dst_v.at[j]],
                                        add=True)

                        @pl.when(j + 2 < _EPT)
                        def _():
                            pltpu.async_copy(
                                table.at[src_v.at[j + 2]], bufs[b], sems[b])
                    return carry

                lax.fori_loop(0, _EPT // 2, cbody, 0)
            return go

        pl.when(c == 0)(_mk_gather(tables[2 * p]))
        pl.when(c == 1)(_mk_gather(tables[2 * p + 1]))
        plsc.subcore_barrier()

        def _mk_out(o):
            def go():
                pltpu.sync_copy(shared.at[pl.ds(s * _RPT, _RPT)],
                                o.at[pl.ds(s * _RPT, _RPT)])
            return go

        pl.when(c == 0)(_mk_out(outs[2 * p]))
        pl.when(c == 1)(_mk_out(outs[2 * p + 1]))
        plsc.subcore_barrier()

    pltpu.sync_copy(deg_v, degpr.at[wid])


def _sc_aggregate(x0, x1, x2, x3, src2, dst2, zrows, zdeg):
    f32 = jnp.float32
    out_type = [jax.ShapeDtypeStruct((_BN, _Q), f32) for _ in range(4)]
    out_type.append(jax.ShapeDtypeStruct((_NC * _NS, _BN // 128, 128), f32))
    kern = pl.kernel(
        _sc_body,
        out_type=out_type,
        mesh=plsc.VectorSubcoreMesh(core_axis_name="c", subcore_axis_name="s"),
        compiler_params=pltpu.CompilerParams(needs_layout_passes=False,
                                              use_tc_tiling_on_sc=False),
        scratch_types=[
            pltpu.VMEM((_EPT, _CH), jnp.int32),
            pltpu.VMEM((_EPT, _CH), jnp.int32),
            pltpu.VMEM((_CH, _Q), f32),
            pltpu.VMEM((_CH, _Q), f32),
            pltpu.VMEM((_BN // 128, 128), f32),
            pltpu.VMEM_SHARED((_BN, _Q), f32),
            pltpu.SemaphoreType.DMA,
            pltpu.SemaphoreType.DMA,
        ],
    )
    return kern(x0, x1, x2, x3, src2, dst2, zrows, zdeg)


def _softmax_mix_log(logits, valid, exp_temp, rand_coef):
    neg = jnp.float32(-1e9)
    masked = jnp.where(valid > 0, logits / exp_temp, neg)
    m = jnp.max(masked, axis=-1, keepdims=True)
    e = jnp.exp(masked - m)
    p = e / jnp.sum(e, axis=-1, keepdims=True)
    uni = valid / jnp.clip(jnp.sum(valid, axis=-1, keepdims=True), 1.0, None)
    probs = (1.0 - rand_coef) * p + rand_coef * uni
    return jnp.log(probs + 1e-20)


def _tc_main_body(xr, ar, dgr, mr, wgr, bgr, w1dr, b1dr, w2dr, w1br, b1br,
                  w2br, sclr, hr, lpdr, lbwr):
    exp_temp = sclr[0, 0]
    rand_coef = sclr[0, 1]
    b_dep2 = sclr[0, 2]
    b_bw2 = sclr[0, 3]
    deg = jnp.clip(0.5 * jnp.sum(dgr[...], axis=0), 1.0, None)
    x = xr[...] + ar[...] / deg[:, None]
    h = jnp.maximum(jnp.dot(x, wgr[...], preferred_element_type=jnp.float32)
                    + bgr[...], 0.0)
    hr[...] = h
    hd = jnp.maximum(jnp.dot(h, w1dr[...], preferred_element_type=jnp.float32)
                     + b1dr[...], 0.0)
    ld = jnp.sum(hd * w2dr[...], axis=1).reshape(4, _N) + b_dep2
    hb = jnp.maximum(jnp.dot(h, w1br[...], preferred_element_type=jnp.float32)
                     + b1br[...], 0.0)
    lb = jnp.sum(hb * w2br[...], axis=1).reshape(4, _N) + b_bw2
    dep_mask = jnp.max(mr[...], axis=1)
    lpdr[...] = _softmax_mix_log(ld, dep_mask, exp_temp, rand_coef)[None]
    col = lax.broadcasted_iota(jnp.int32, (4, _N), 1)
    bmask = (dep_mask <= 0.0) & (col > 0)
    lbwr[...] = jnp.where(bmask, lb, jnp.float32(-1e9))[None]


def _tc_main(x, agg, degp, mask_f, W_gcn, b_gcn, W_dep1, b_dep1, w_dep2,
             W_bw1, b_bw1, w_bw2, scal):
    f32 = jnp.float32
    grid = _BN // 512
    return pl.pallas_call(
        _tc_main_body,
        grid=(grid,),
        in_specs=[
            pl.BlockSpec((512, _D), lambda i: (i, 0)),
            pl.BlockSpec((512, _D), lambda i: (i, 0)),
            pl.BlockSpec((_NC * _NS, 512), lambda i: (0, i)),
            pl.BlockSpec((4, _N, _N), lambda i: (i, 0, 0)),
            pl.BlockSpec((_D, _D), lambda i: (0, 0)),
            pl.BlockSpec((1, _D), lambda i: (0, 0)),
            pl.BlockSpec((_D, 512), lambda i: (0, 0)),
            pl.BlockSpec((1, 512), lambda i: (0, 0)),
            pl.BlockSpec((1, 512), lambda i: (0, 0)),
            pl.BlockSpec((_D, 512), lambda i: (0, 0)),
            pl.BlockSpec((1, 512), lambda i: (0, 0)),
            pl.BlockSpec((1, 512), lambda i: (0, 0)),
            pl.BlockSpec(memory_space=pltpu.SMEM),
        ],
        out_specs=[
            pl.BlockSpec((512, _D), lambda i: (i, 0)),
            pl.BlockSpec((1, 4, _N), lambda i: (i, 0, 0)),
            pl.BlockSpec((1, 4, _N), lambda i: (i, 0, 0)),
        ],
        out_shape=[
            jax.ShapeDtypeStruct((_BN, _D), f32),
            jax.ShapeDtypeStruct((_B // 4, 4, _N), f32),
            jax.ShapeDtypeStruct((_B // 4, 4, _N), f32),
        ],
    )(x, agg, degp, mask_f, W_gcn, b_gcn, W_dep1, b_dep1, w_dep2, W_bw1,
      b_bw1, w_bw2, scal)


def _tc_head_body(hr, mr, wr, ur, vr, lpdr, depr, sclr, lphr, lpfdr):
    b = pl.program_id(0)
    exp_temp = sclr[0, 0]
    rand_coef = sclr[0, 1]
    s_bi = sclr[0, 2]
    d = depr[b]
    hblk = hr[0]
    oh = (lax.broadcasted_iota(jnp.int32, (1, _N), 1) == d).astype(jnp.float32)
    hd = jnp.dot(oh, hblk, preferred_element_type=jnp.float32)
    v = lax.dot_general(hd, wr[...], (((1,), (1,)), ((), ())),
                        preferred_element_type=jnp.float32) + ur[...]
    t = jnp.sum(vr[...] * hd) + s_bi
    lh = lax.dot_general(v, hblk, (((1,), (1,)), ((), ())),
                         preferred_element_type=jnp.float32) + t
    hm = lax.dot_general(mr[0], oh, (((1,), (1,)), ((), ())),
                         preferred_element_type=jnp.float32).reshape(1, _N)
    lphr[...] = _softmax_mix_log(lh, hm, exp_temp, rand_coef)[None]
    lpfdr[...] = jnp.sum(lpdr[0] * oh, axis=-1, keepdims=True)[None]


def _tc_head(h3, mask_f, Wc, u_bi, w_bi, logp_dep, dep_ids, scal):
    f32 = jnp.float32
    return pl.pallas_call(
        _tc_head_body,
        grid=(_B,),
        in_specs=[
            pl.BlockSpec((1, _N, _D), lambda b: (b, 0, 0)),
            pl.BlockSpec((1, _N, _N), lambda b: (b, 0, 0)),
            pl.BlockSpec((_D, _D), lambda b: (0, 0)),
            pl.BlockSpec((1, _D), lambda b: (0, 0)),
            pl.BlockSpec((1, _D), lambda b: (0, 0)),
            pl.BlockSpec((1, 1, _N), lambda b: (b, 0, 0)),
            pl.BlockSpec(memory_space=pltpu.SMEM),
            pl.BlockSpec(memory_space=pltpu.SMEM),
        ],
        out_specs=[
            pl.BlockSpec((1, 1, _N), lambda b: (b, 0, 0)),
            pl.BlockSpec((1, 1, 1), lambda b: (b, 0, 0)),
        ],
        out_shape=[
            jax.ShapeDtypeStruct((_B, 1, _N), f32),
            jax.ShapeDtypeStruct((_B, 1, 1), f32),
        ],
    )(h3, mask_f, Wc, u_bi, w_bi, logp_dep.reshape(_B, 1, _N), dep_ids, scal)


def kernel(node_feats, edge_index, mask, exp_temp, rand_coef, W_gcn, b_gcn,
           W_dep1, b_dep1, W_dep2, b_dep2, W_bi, W_bw1, b_bw1, W_bw2, b_bw2):
    f32 = jnp.float32
    x = node_feats.astype(f32)
    src2 = edge_index[0].reshape(_E // _CH, _CH).astype(jnp.int32)
    dst2 = edge_index[1].reshape(_E // _CH, _CH).astype(jnp.int32)
    mask_f = mask.astype(f32)

    xq = [x[:, i * _Q:(i + 1) * _Q] for i in range(4)]
    zrows = jnp.zeros((_BN, _Q), f32)
    zdeg = jnp.zeros((_BN // 128, 128), f32)
    o0, o1, o2, o3, degp = _sc_aggregate(xq[0], xq[1], xq[2], xq[3], src2,
                                         dst2, zrows, zdeg)
    agg = jnp.concatenate([o0, o1, o2, o3], axis=1)
    degp2 = degp.reshape(_NC * _NS, _BN)

    et = jnp.asarray(exp_temp, f32)
    rc = jnp.asarray(rand_coef, f32)
    scal1 = jnp.stack([et, rc, b_dep2.reshape(()).astype(f32),
                       b_bw2.reshape(()).astype(f32)]).reshape(1, 4)
    h, logp_dep, logits_bw = _tc_main(
        x, agg, degp2, mask_f, W_gcn, b_gcn.reshape(1, _D),
        W_dep1, b_dep1.reshape(1, 512), W_dep2.reshape(1, 512),
        W_bw1, b_bw1.reshape(1, 512), W_bw2.reshape(1, 512), scal1)
    logp_dep = logp_dep.reshape(_B, _N)
    logits_bw = logits_bw.reshape(_B, _N)

    dep_ids = jax.random.categorical(jax.random.key(1), logp_dep,
                                     axis=-1)[:, None]

    Wc = W_bi[:_D, :_D]
    u_bi = W_bi[:_D, _D].reshape(1, _D)
    w_bi = W_bi[_D, :_D].reshape(1, _D)
    scal2 = jnp.stack([et, rc, W_bi[_D, _D].astype(f32)]).reshape(1, 3)
    logp_head, log_pF_dep = _tc_head(
        h.reshape(_B, _N, _D), mask_f, Wc, u_bi, w_bi, logp_dep,
        dep_ids[:, 0].astype(jnp.int32), scal2)
    logp_head = logp_head.reshape(_B, _N)
    log_pF_dep = log_pF_dep.reshape(_B, 1)

    head_ids = jax.random.categorical(jax.random.key(2), logp_head,
                                      axis=-1)[:, None]
    log_pF_head = jnp.take_along_axis(logp_head, head_ids, axis=-1)
    return (head_ids, dep_ids), (log_pF_head, log_pF_dep), logits_bw


# speedup vs baseline: 3.5647x; 1.0306x over previous
import functools

import jax
import jax.numpy as jnp
from jax import lax
from jax.experimental import pallas as pl
from jax.experimental.pallas import tpu as pltpu
from jax.experimental.pallas import tpu_sc as plsc

_B, _N, _D, _E = 32, 128, 1024, 65536
_BN = _B * _N
_Q = _D // 4
_NS = 16
_NC = 2
_CH = 32
_EPT = _E // (_NS * _CH)
_RPT = _BN // _NS


def _sc_body(x0r, x1r, x2r, x3r, srcr, dstr, zr, zdr,
             o0r, o1r, o2r, o3r, degpr,
             src_v, dst_v, rows_v, rows_w, deg_v, shared, sem, sem2):
    c = lax.axis_index("c")
    s = lax.axis_index("s")
    wid = s * _NC + c
    pltpu.sync_copy(srcr.at[pl.ds(s * _EPT, _EPT)], src_v)
    pltpu.sync_copy(dstr.at[pl.ds(s * _EPT, _EPT)], dst_v)
    pltpu.sync_copy(zdr, deg_v)

    ones16 = jnp.ones((16,), jnp.float32)

    def dbody(i, carry):
        d16 = dst_v[i >> 1, pl.ds((i & 1) * 16, 16)]
        plsc.addupdate_scatter(
            deg_v,
            [lax.shift_right_logical(d16, 7), lax.bitwise_and(d16, 127)],
            ones16)
        return carry

    lax.fori_loop(0, _EPT * (_CH // 16), dbody, 0)

    tables = [x0r, x1r, x2r, x3r]
    outs = [o0r, o1r, o2r, o3r]
    for p in range(2):
        pltpu.sync_copy(zr.at[pl.ds(s * _RPT, _RPT)],
                        shared.at[pl.ds(s * _RPT, _RPT)])
        plsc.subcore_barrier()

        def _mk_gather(table):
            def go():
                bufs = (rows_v, rows_w)
                sems = (sem, sem2)
                for b in range(2):
                    pltpu.async_copy(table.at[src_v.at[b]], bufs[b], sems[b])

                def cbody(i, carry):
                    j0 = i * 2
                    for b in range(2):
                        j = j0 + b
                        pltpu.make_async_copy(
                            table.at[src_v.at[j]], bufs[b], sems[b]).wait()
                        pltpu.sync_copy(bufs[b], shared.at[dst_v.at[j]],
                                        add=True)

                        @pl.when(j + 2 < _EPT)
                        def _():
                            pltpu.async_copy(
                                table.at[src_v.at[j + 2]], bufs[b], sems[b])
                    return carry

                lax.fori_loop(0, _EPT // 2, cbody, 0)
            return go

        pl.when(c == 0)(_mk_gather(tables[2 * p]))
        pl.when(c == 1)(_mk_gather(tables[2 * p + 1]))
        plsc.subcore_barrier()

        def _mk_out(o):
            def go():
                pltpu.sync_copy(shared.at[pl.ds(s * _RPT, _RPT)],
                                o.at[pl.ds(s * _RPT, _RPT)])
            return go

        pl.when(c == 0)(_mk_out(outs[2 * p]))
        pl.when(c == 1)(_mk_out(outs[2 * p + 1]))
        plsc.subcore_barrier()

    pltpu.sync_copy(deg_v, degpr.at[wid])


def _sc_aggregate(x0, x1, x2, x3, src2, dst2, zrows, zdeg):
    f32 = jnp.float32
    out_type = [jax.ShapeDtypeStruct((_BN, _Q), f32) for _ in range(4)]
    out_type.append(jax.ShapeDtypeStruct((_NC * _NS, _BN // 128, 128), f32))
    kern = pl.kernel(
        _sc_body,
        out_type=out_type,
        mesh=plsc.VectorSubcoreMesh(core_axis_name="c", subcore_axis_name="s"),
        compiler_params=pltpu.CompilerParams(needs_layout_passes=False,
                                              use_tc_tiling_on_sc=False),
        scratch_types=[
            pltpu.VMEM((_EPT, _CH), jnp.int32),
            pltpu.VMEM((_EPT, _CH), jnp.int32),
            pltpu.VMEM((_CH, _Q), f32),
            pltpu.VMEM((_CH, _Q), f32),
            pltpu.VMEM((_BN // 128, 128), f32),
            pltpu.VMEM_SHARED((_BN, _Q), f32),
            pltpu.SemaphoreType.DMA,
            pltpu.SemaphoreType.DMA,
        ],
    )
    return kern(x0, x1, x2, x3, src2, dst2, zrows, zdeg)


def _softmax_mix_log(logits, valid, exp_temp, rand_coef):
    neg = jnp.float32(-1e9)
    masked = jnp.where(valid > 0, logits / exp_temp, neg)
    m = jnp.max(masked, axis=-1, keepdims=True)
    e = jnp.exp(masked - m)
    p = e / jnp.sum(e, axis=-1, keepdims=True)
    uni = valid / jnp.clip(jnp.sum(valid, axis=-1, keepdims=True), 1.0, None)
    probs = (1.0 - rand_coef) * p + rand_coef * uni
    return jnp.log(probs + 1e-20)


def _tc_main_body(xr, ar, dgr, mr, wgr, bgr, w1dr, b1dr, w2dr, w1br, b1br,
                  w2br, sclr, hr, lpdr, lbwr):
    exp_temp = sclr[0, 0]
    rand_coef = sclr[0, 1]
    b_dep2 = sclr[0, 2]
    b_bw2 = sclr[0, 3]
    b16 = lambda a: a.astype(jnp.bfloat16)
    f = jnp.float32
    deg = jnp.clip(0.5 * jnp.sum(dgr[...], axis=0), 1.0, None)
    x = xr[...] + ar[...] / deg[:, None]
    h = jnp.maximum(jnp.dot(b16(x), b16(wgr[...]), preferred_element_type=f)
                    + bgr[...], 0.0)
    hr[...] = h
    hd = jnp.maximum(jnp.dot(b16(h), b16(w1dr[...]), preferred_element_type=f)
                     + b1dr[...], 0.0)
    ld = jnp.sum(b16(hd).astype(f) * b16(w2dr[...]).astype(f),
                 axis=1).reshape(4, _N) + b_dep2
    hb = jnp.maximum(jnp.dot(b16(h), b16(w1br[...]), preferred_element_type=f)
                     + b1br[...], 0.0)
    lb = jnp.sum(b16(hb).astype(f) * b16(w2br[...]).astype(f),
                 axis=1).reshape(4, _N) + b_bw2
    dep_mask = jnp.max(mr[...], axis=1)
    lpdr[...] = _softmax_mix_log(ld, dep_mask, exp_temp, rand_coef)[None]
    col = lax.broadcasted_iota(jnp.int32, (4, _N), 1)
    bmask = (dep_mask <= 0.0) & (col > 0)
    lbwr[...] = jnp.where(bmask, lb, jnp.float32(-1e9))[None]


def _tc_main(x, agg, degp, mask_f, W_gcn, b_gcn, W_dep1, b_dep1, w_dep2,
             W_bw1, b_bw1, w_bw2, scal):
    f32 = jnp.float32
    grid = _BN // 512
    return pl.pallas_call(
        _tc_main_body,
        grid=(grid,),
        in_specs=[
            pl.BlockSpec((512, _D), lambda i: (i, 0)),
            pl.BlockSpec((512, _D), lambda i: (i, 0)),
            pl.BlockSpec((_NC * _NS, 512), lambda i: (0, i)),
            pl.BlockSpec((4, _N, _N), lambda i: (i, 0, 0)),
            pl.BlockSpec((_D, _D), lambda i: (0, 0)),
            pl.BlockSpec((1, _D), lambda i: (0, 0)),
            pl.BlockSpec((_D, 512), lambda i: (0, 0)),
            pl.BlockSpec((1, 512), lambda i: (0, 0)),
            pl.BlockSpec((1, 512), lambda i: (0, 0)),
            pl.BlockSpec((_D, 512), lambda i: (0, 0)),
            pl.BlockSpec((1, 512), lambda i: (0, 0)),
            pl.BlockSpec((1, 512), lambda i: (0, 0)),
            pl.BlockSpec(memory_space=pltpu.SMEM),
        ],
        out_specs=[
            pl.BlockSpec((512, _D), lambda i: (i, 0)),
            pl.BlockSpec((1, 4, _N), lambda i: (i, 0, 0)),
            pl.BlockSpec((1, 4, _N), lambda i: (i, 0, 0)),
        ],
        out_shape=[
            jax.ShapeDtypeStruct((_BN, _D), f32),
            jax.ShapeDtypeStruct((_B // 4, 4, _N), f32),
            jax.ShapeDtypeStruct((_B // 4, 4, _N), f32),
        ],
    )(x, agg, degp, mask_f, W_gcn, b_gcn, W_dep1, b_dep1, w_dep2, W_bw1,
      b_bw1, w_bw2, scal)


def _tc_head_body(hr, mr, wr, ur, vr, lpdr, depr, sclr, lphr, lpfdr):
    b = pl.program_id(0)
    exp_temp = sclr[0, 0]
    rand_coef = sclr[0, 1]
    s_bi = sclr[0, 2]
    b16 = lambda a: a.astype(jnp.bfloat16)
    f = jnp.float32
    d = depr[b]
    hblk = hr[0]
    oh = (lax.broadcasted_iota(jnp.int32, (1, _N), 1) == d).astype(f)
    hd = jnp.sum(hblk * oh.reshape(_N, 1), axis=0, keepdims=True)
    t1a = jnp.dot(b16(hblk), b16(wr[...]), preferred_element_type=f)         + b16(ur[...]).astype(f)
    t1b = jnp.sum(b16(hblk).astype(f) * b16(vr[...]).astype(f), axis=1,
                  keepdims=True) + b16(jnp.float32(s_bi)).astype(f)
    lh = (jnp.sum(t1a * hd, axis=1, keepdims=True) + t1b).reshape(1, _N)
    hm = jnp.sum(mr[0] * oh, axis=1, keepdims=True).reshape(1, _N)
    lphr[...] = _softmax_mix_log(lh, hm, exp_temp, rand_coef)[None]
    lpfdr[...] = jnp.sum(lpdr[0] * oh, axis=-1, keepdims=True)[None]


def _tc_head(h3, mask_f, Wc, u_bi, w_bi, logp_dep, dep_ids, scal):
    f32 = jnp.float32
    return pl.pallas_call(
        _tc_head_body,
        grid=(_B,),
        in_specs=[
            pl.BlockSpec((1, _N, _D), lambda b: (b, 0, 0)),
            pl.BlockSpec((1, _N, _N), lambda b: (b, 0, 0)),
            pl.BlockSpec((_D, _D), lambda b: (0, 0)),
            pl.BlockSpec((1, _D), lambda b: (0, 0)),
            pl.BlockSpec((1, _D), lambda b: (0, 0)),
            pl.BlockSpec((1, 1, _N), lambda b: (b, 0, 0)),
            pl.BlockSpec(memory_space=pltpu.SMEM),
            pl.BlockSpec(memory_space=pltpu.SMEM),
        ],
        out_specs=[
            pl.BlockSpec((1, 1, _N), lambda b: (b, 0, 0)),
            pl.BlockSpec((1, 1, 1), lambda b: (b, 0, 0)),
        ],
        out_shape=[
            jax.ShapeDtypeStruct((_B, 1, _N), f32),
            jax.ShapeDtypeStruct((_B, 1, 1), f32),
        ],
    )(h3, mask_f, Wc, u_bi, w_bi, logp_dep.reshape(_B, 1, _N), dep_ids, scal)


def kernel(node_feats, edge_index, mask, exp_temp, rand_coef, W_gcn, b_gcn,
           W_dep1, b_dep1, W_dep2, b_dep2, W_bi, W_bw1, b_bw1, W_bw2, b_bw2):
    f32 = jnp.float32
    x = node_feats.astype(f32)
    src2 = edge_index[0].reshape(_E // _CH, _CH).astype(jnp.int32)
    dst2 = edge_index[1].reshape(_E // _CH, _CH).astype(jnp.int32)
    mask_f = mask.astype(f32)

    xq = [x[:, i * _Q:(i + 1) * _Q] for i in range(4)]
    zrows = jnp.zeros((_BN, _Q), f32)
    zdeg = jnp.zeros((_BN // 128, 128), f32)
    o0, o1, o2, o3, degp = _sc_aggregate(xq[0], xq[1], xq[2], xq[3], src2,
                                         dst2, zrows, zdeg)
    agg = jnp.concatenate([o0, o1, o2, o3], axis=1)
    degp2 = degp.reshape(_NC * _NS, _BN)

    et = jnp.asarray(exp_temp, f32)
    rc = jnp.asarray(rand_coef, f32)
    scal1 = jnp.stack([et, rc, b_dep2.reshape(()).astype(f32),
                       b_bw2.reshape(()).astype(f32)]).reshape(1, 4)
    h, logp_dep, logits_bw = _tc_main(
        x, agg, degp2, mask_f, W_gcn, b_gcn.reshape(1, _D),
        W_dep1, b_dep1.reshape(1, 512), W_dep2.reshape(1, 512),
        W_bw1, b_bw1.reshape(1, 512), W_bw2.reshape(1, 512), scal1)
    logp_dep = logp_dep.reshape(_B, _N)
    logits_bw = logits_bw.reshape(_B, _N)

    dep_ids = jax.random.categorical(jax.random.key(1), logp_dep,
                                     axis=-1)[:, None]

    Wc = W_bi[:_D, :_D]
    u_bi = W_bi[:_D, _D].reshape(1, _D)
    w_bi = W_bi[_D, :_D].reshape(1, _D)
    scal2 = jnp.stack([et, rc, W_bi[_D, _D].astype(f32)]).reshape(1, 3)
    logp_head, log_pF_dep = _tc_head(
        h.reshape(_B, _N, _D), mask_f, Wc, w_bi, u_bi, logp_dep,
        dep_ids[:, 0].astype(jnp.int32), scal2)
    logp_head = logp_head.reshape(_B, _N)
    log_pF_dep = log_pF_dep.reshape(_B, 1)

    head_ids = jax.random.categorical(jax.random.key(2), logp_head,
                                      axis=-1)[:, None]
    log_pF_head = jnp.take_along_axis(logp_head, head_ids, axis=-1)
    return (head_ids, dep_ids), (log_pF_head, log_pF_dep), logits_bw


# speedup vs baseline: 3.7671x; 1.0568x over previous
import functools

import jax
import jax.numpy as jnp
from jax import lax
from jax.experimental import pallas as pl
from jax.experimental.pallas import tpu as pltpu
from jax.experimental.pallas import tpu_sc as plsc

_B, _N, _D, _E = 32, 128, 1024, 65536
_BN = _B * _N
_Q = _D // 4
_NS = 16
_NC = 2
_CH = 32
_EPT = _E // (_NS * _CH)
_RPT = _BN // _NS


def _sc_body(xvr, srcr, dstr, zr, zdr,
             o0r, o1r, o2r, o3r, degpr,
             src_v, dst_v, idx_a, idx_b, rows_v, rows_w, deg_v, shared,
             sem, sem2):
    c = lax.axis_index("c")
    s = lax.axis_index("s")
    wid = s * _NC + c
    pltpu.sync_copy(srcr.at[pl.ds(s * _EPT, _EPT)], src_v)
    pltpu.sync_copy(dstr.at[pl.ds(s * _EPT, _EPT)], dst_v)
    pltpu.sync_copy(zdr, deg_v)

    def ibody(i, carry):
        r = i >> 1
        k = (i & 1) * 16
        v = src_v[r, pl.ds(k, 16)] * 4 + c
        idx_a[r, pl.ds(k, 16)] = v
        idx_b[r, pl.ds(k, 16)] = v + 2
        return carry

    lax.fori_loop(0, _EPT * (_CH // 16), ibody, 0)

    ones16 = jnp.ones((16,), jnp.float32)

    def dbody(i, carry):
        d16 = dst_v[i >> 1, pl.ds((i & 1) * 16, 16)]
        plsc.addupdate_scatter(
            deg_v,
            [lax.shift_right_logical(d16, 7), lax.bitwise_and(d16, 127)],
            ones16)
        return carry

    lax.fori_loop(0, _EPT * (_CH // 16), dbody, 0)

    outs = [o0r, o1r, o2r, o3r]
    for p, idx in ((0, idx_a), (1, idx_b)):
        pltpu.sync_copy(zr.at[pl.ds(s * _RPT, _RPT)],
                        shared.at[pl.ds(s * _RPT, _RPT)])
        plsc.subcore_barrier()

        bufs = (rows_v, rows_w)
        sems = (sem, sem2)
        for b in range(2):
            pltpu.async_copy(xvr.at[idx.at[b]], bufs[b], sems[b])

        def cbody(i, carry, idx=idx):
            j0 = i * 2
            for b in range(2):
                j = j0 + b
                pltpu.make_async_copy(
                    xvr.at[idx.at[j]], bufs[b], sems[b]).wait()
                pltpu.sync_copy(bufs[b], shared.at[dst_v.at[j]], add=True)

                @pl.when(j + 2 < _EPT)
                def _():
                    pltpu.async_copy(xvr.at[idx.at[j + 2]], bufs[b], sems[b])
            return carry

        lax.fori_loop(0, _EPT // 2, cbody, 0)
        plsc.subcore_barrier()

        def _mk_out(o):
            def go():
                pltpu.sync_copy(shared.at[pl.ds(s * _RPT, _RPT)],
                                o.at[pl.ds(s * _RPT, _RPT)])
            return go

        pl.when(c == 0)(_mk_out(outs[2 * p]))
        pl.when(c == 1)(_mk_out(outs[2 * p + 1]))
        plsc.subcore_barrier()

    pltpu.sync_copy(deg_v, degpr.at[wid])


def _sc_aggregate(xv, src2, dst2, zrows, zdeg):
    f32 = jnp.float32
    out_type = [jax.ShapeDtypeStruct((_BN, _Q), f32) for _ in range(4)]
    out_type.append(jax.ShapeDtypeStruct((_NC * _NS, _BN // 128, 128), f32))
    kern = pl.kernel(
        _sc_body,
        out_type=out_type,
        mesh=plsc.VectorSubcoreMesh(core_axis_name="c", subcore_axis_name="s"),
        compiler_params=pltpu.CompilerParams(needs_layout_passes=False,
                                              use_tc_tiling_on_sc=False),
        scratch_types=[
            pltpu.VMEM((_EPT, _CH), jnp.int32),
            pltpu.VMEM((_EPT, _CH), jnp.int32),
            pltpu.VMEM((_EPT, _CH), jnp.int32),
            pltpu.VMEM((_EPT, _CH), jnp.int32),
            pltpu.VMEM((_CH, _Q), f32),
            pltpu.VMEM((_CH, _Q), f32),
            pltpu.VMEM((_BN // 128, 128), f32),
            pltpu.VMEM_SHARED((_BN, _Q), f32),
            pltpu.SemaphoreType.DMA,
            pltpu.SemaphoreType.DMA,
        ],
    )
    return kern(xv, src2, dst2, zrows, zdeg)


def _softmax_mix_log(logits, valid, exp_temp, rand_coef):
    neg = jnp.float32(-1e9)
    masked = jnp.where(valid > 0, logits / exp_temp, neg)
    m = jnp.max(masked, axis=-1, keepdims=True)
    e = jnp.exp(masked - m)
    p = e / jnp.sum(e, axis=-1, keepdims=True)
    uni = valid / jnp.clip(jnp.sum(valid, axis=-1, keepdims=True), 1.0, None)
    probs = (1.0 - rand_coef) * p + rand_coef * uni
    return jnp.log(probs + 1e-20)


def _tc_main_body(xr, a0r, a1r, a2r, a3r, dgr, mr, wgr, bgr, w1dr, b1dr,
                  w2dr, w1br, b1br, w2br, sclr, hr, lpdr, lbwr):
    exp_temp = sclr[0, 0]
    rand_coef = sclr[0, 1]
    b_dep2 = sclr[0, 2]
    b_bw2 = sclr[0, 3]
    b16 = lambda a: a.astype(jnp.bfloat16)
    f = jnp.float32
    deg = jnp.clip(0.5 * jnp.sum(dgr[...], axis=0), 1.0, None)
    a = jnp.concatenate([a0r[...], a1r[...], a2r[...], a3r[...]], axis=1)
    x = xr[...] + a / deg[:, None]
    h = jnp.maximum(jnp.dot(b16(x), b16(wgr[...]), preferred_element_type=f)
                    + bgr[...], 0.0)
    hr[...] = h
    hd = jnp.maximum(jnp.dot(b16(h), b16(w1dr[...]), preferred_element_type=f)
                     + b1dr[...], 0.0)
    ld = jnp.sum(b16(hd).astype(f) * b16(w2dr[...]).astype(f),
                 axis=1).reshape(4, _N) + b_dep2
    hb = jnp.maximum(jnp.dot(b16(h), b16(w1br[...]), preferred_element_type=f)
                     + b1br[...], 0.0)
    lb = jnp.sum(b16(hb).astype(f) * b16(w2br[...]).astype(f),
                 axis=1).reshape(4, _N) + b_bw2
    dep_mask = jnp.max(mr[...], axis=1)
    lpdr[...] = _softmax_mix_log(ld, dep_mask, exp_temp, rand_coef)[None]
    col = lax.broadcasted_iota(jnp.int32, (4, _N), 1)
    bmask = (dep_mask <= 0.0) & (col > 0)
    lbwr[...] = jnp.where(bmask, lb, jnp.float32(-1e9))[None]


def _tc_main(x, a0, a1, a2, a3, degp, mask_f, W_gcn, b_gcn, W_dep1, b_dep1,
             w_dep2, W_bw1, b_bw1, w_bw2, scal):
    f32 = jnp.float32
    grid = _BN // 512
    return pl.pallas_call(
        _tc_main_body,
        grid=(grid,),
        in_specs=[
            pl.BlockSpec((512, _D), lambda i: (i, 0)),
            pl.BlockSpec((512, _Q), lambda i: (i, 0)),
            pl.BlockSpec((512, _Q), lambda i: (i, 0)),
            pl.BlockSpec((512, _Q), lambda i: (i, 0)),
            pl.BlockSpec((512, _Q), lambda i: (i, 0)),
            pl.BlockSpec((_NC * _NS, 512), lambda i: (0, i)),
            pl.BlockSpec((4, _N, _N), lambda i: (i, 0, 0)),
            pl.BlockSpec((_D, _D), lambda i: (0, 0)),
            pl.BlockSpec((1, _D), lambda i: (0, 0)),
            pl.BlockSpec((_D, 512), lambda i: (0, 0)),
            pl.BlockSpec((1, 512), lambda i: (0, 0)),
            pl.BlockSpec((1, 512), lambda i: (0, 0)),
            pl.BlockSpec((_D, 512), lambda i: (0, 0)),
            pl.BlockSpec((1, 512), lambda i: (0, 0)),
            pl.BlockSpec((1, 512), lambda i: (0, 0)),
            pl.BlockSpec(memory_space=pltpu.SMEM),
        ],
        out_specs=[
            pl.BlockSpec((512, _D), lambda i: (i, 0)),
            pl.BlockSpec((1, 4, _N), lambda i: (i, 0, 0)),
            pl.BlockSpec((1, 4, _N), lambda i: (i, 0, 0)),
        ],
        out_shape=[
            jax.ShapeDtypeStruct((_BN, _D), f32),
            jax.ShapeDtypeStruct((_B // 4, 4, _N), f32),
            jax.ShapeDtypeStruct((_B // 4, 4, _N), f32),
        ],
    )(x, a0, a1, a2, a3, degp, mask_f, W_gcn, b_gcn, W_dep1, b_dep1, w_dep2,
      W_bw1, b_bw1, w_bw2, scal)


def _tc_head_body(hr, mr, wr, ur, vr, lpdr, depr, sclr, lphr, lpfdr):
    b = pl.program_id(0)
    exp_temp = sclr[0, 0]
    rand_coef = sclr[0, 1]
    s_bi = sclr[0, 2]
    b16 = lambda a: a.astype(jnp.bfloat16)
    f = jnp.float32
    d = depr[b]
    hblk = hr[0]
    oh = (lax.broadcasted_iota(jnp.int32, (1, _N), 1) == d).astype(f)
    hd = jnp.sum(hblk * oh.reshape(_N, 1), axis=0, keepdims=True)
    t1a = jnp.dot(b16(hblk), b16(wr[...]), preferred_element_type=f)         + b16(ur[...]).astype(f)
    t1b = jnp.sum(b16(hblk).astype(f) * b16(vr[...]).astype(f), axis=1,
                  keepdims=True) + b16(jnp.float32(s_bi)).astype(f)
    lh = (jnp.sum(t1a * hd, axis=1, keepdims=True) + t1b).reshape(1, _N)
    hm = jnp.sum(mr[0] * oh, axis=1, keepdims=True).reshape(1, _N)
    lphr[...] = _softmax_mix_log(lh, hm, exp_temp, rand_coef)[None]
    lpfdr[...] = jnp.sum(lpdr[0] * oh, axis=-1, keepdims=True)[None]


def _tc_head(h3, mask_f, Wc, u_bi, w_bi, logp_dep, dep_ids, scal):
    f32 = jnp.float32
    return pl.pallas_call(
        _tc_head_body,
        grid=(_B,),
        in_specs=[
            pl.BlockSpec((1, _N, _D), lambda b: (b, 0, 0)),
            pl.BlockSpec((1, _N, _N), lambda b: (b, 0, 0)),
            pl.BlockSpec((_D, _D), lambda b: (0, 0)),
            pl.BlockSpec((1, _D), lambda b: (0, 0)),
            pl.BlockSpec((1, _D), lambda b: (0, 0)),
            pl.BlockSpec((1, 1, _N), lambda b: (b, 0, 0)),
            pl.BlockSpec(memory_space=pltpu.SMEM),
            pl.BlockSpec(memory_space=pltpu.SMEM),
        ],
        out_specs=[
            pl.BlockSpec((1, 1, _N), lambda b: (b, 0, 0)),
            pl.BlockSpec((1, 1, 1), lambda b: (b, 0, 0)),
        ],
        out_shape=[
            jax.ShapeDtypeStruct((_B, 1, _N), f32),
            jax.ShapeDtypeStruct((_B, 1, 1), f32),
        ],
    )(h3, mask_f, Wc, u_bi, w_bi, logp_dep.reshape(_B, 1, _N), dep_ids, scal)


def kernel(node_feats, edge_index, mask, exp_temp, rand_coef, W_gcn, b_gcn,
           W_dep1, b_dep1, W_dep2, b_dep2, W_bi, W_bw1, b_bw1, W_bw2, b_bw2):
    f32 = jnp.float32
    x = node_feats.astype(f32)
    src2 = edge_index[0].reshape(_E // _CH, _CH).astype(jnp.int32)
    dst2 = edge_index[1].reshape(_E // _CH, _CH).astype(jnp.int32)
    mask_f = mask.astype(f32)

    zrows = jnp.zeros((_BN, _Q), f32)
    zdeg = jnp.zeros((_BN // 128, 128), f32)
    o0, o1, o2, o3, degp = _sc_aggregate(x.reshape(_BN * 4, _Q), src2,
                                         dst2, zrows, zdeg)
    degp2 = degp.reshape(_NC * _NS, _BN)

    et = jnp.asarray(exp_temp, f32)
    rc = jnp.asarray(rand_coef, f32)
    scal1 = jnp.stack([et, rc, b_dep2.reshape(()).astype(f32),
                       b_bw2.reshape(()).astype(f32)]).reshape(1, 4)
    h, logp_dep, logits_bw = _tc_main(
        x, o0, o1, o2, o3, degp2, mask_f, W_gcn, b_gcn.reshape(1, _D),
        W_dep1, b_dep1.reshape(1, 512), W_dep2.reshape(1, 512),
        W_bw1, b_bw1.reshape(1, 512), W_bw2.reshape(1, 512), scal1)
    logp_dep = logp_dep.reshape(_B, _N)
    logits_bw = logits_bw.reshape(_B, _N)

    dep_ids = jax.random.categorical(jax.random.key(1), logp_dep,
                                     axis=-1)[:, None]

    Wc = W_bi[:_D, :_D]
    u_bi = W_bi[:_D, _D].reshape(1, _D)
    w_bi = W_bi[_D, :_D].reshape(1, _D)
    scal2 = jnp.stack([et, rc, W_bi[_D, _D].astype(f32)]).reshape(1, 3)
    logp_head, log_pF_dep = _tc_head(
        h.reshape(_B, _N, _D), mask_f, Wc, w_bi, u_bi, logp_dep,
        dep_ids[:, 0].astype(jnp.int32), scal2)
    logp_head = logp_head.reshape(_B, _N)
    log_pF_dep = log_pF_dep.reshape(_B, 1)

    head_ids = jax.random.categorical(jax.random.key(2), logp_head,
                                      axis=-1)[:, None]
    log_pF_head = jnp.take_along_axis(logp_head, head_ids, axis=-1)
    return (head_ids, dep_ids), (log_pF_head, log_pF_dep), logits_bw


# speedup vs baseline: 4.5293x; 1.2023x over previous
import functools

import jax
import jax.numpy as jnp
from jax import lax
from jax.experimental import pallas as pl
from jax.experimental.pallas import tpu as pltpu
from jax.experimental.pallas import tpu_sc as plsc

_B, _N, _D, _E = 32, 128, 1024, 65536
_BN = _B * _N
_Q = _D // 4
_NS = 16
_NC = 2
_CH = 32
_EPT = _E // (_NS * _CH)
_RPT = _BN // _NS


def _sc_body(xvr, srcr, dstr, zr, zdr,
             o0r, o1r, o2r, o3r, degpr,
             src_v, dst_v, idx_a, idx_b, r0, r1, r2, r3, deg_v, shared,
             g0, g1, g2, g3, s0, s1, s2, s3):
    c = lax.axis_index("c")
    s = lax.axis_index("s")
    wid = s * _NC + c
    pltpu.sync_copy(srcr.at[pl.ds(s * _EPT, _EPT)], src_v)
    pltpu.sync_copy(dstr.at[pl.ds(s * _EPT, _EPT)], dst_v)
    pltpu.sync_copy(zdr, deg_v)

    def ibody(i, carry):
        r = i >> 1
        k = (i & 1) * 16
        v = src_v[r, pl.ds(k, 16)] * 4 + c
        idx_a[r, pl.ds(k, 16)] = v
        idx_b[r, pl.ds(k, 16)] = v + 2
        return carry

    lax.fori_loop(0, _EPT * (_CH // 16), ibody, 0)

    ones16 = jnp.ones((16,), jnp.float32)

    def dbody(i, carry):
        d16 = dst_v[i >> 1, pl.ds((i & 1) * 16, 16)]
        plsc.addupdate_scatter(
            deg_v,
            [lax.shift_right_logical(d16, 7), lax.bitwise_and(d16, 127)],
            ones16)
        return carry

    lax.fori_loop(0, _EPT * (_CH // 16), dbody, 0)

    outs = [o0r, o1r, o2r, o3r]
    for p, idx in ((0, idx_a), (1, idx_b)):
        pltpu.sync_copy(zr.at[pl.ds(s * _RPT, _RPT)],
                        shared.at[pl.ds(s * _RPT, _RPT)])
        plsc.subcore_barrier()

        bufs = (r0, r1, r2, r3)
        gsems = (g0, g1, g2, g3)
        ssems = (s0, s1, s2, s3)
        for b in range(4):
            pltpu.async_copy(xvr.at[idx.at[b]], bufs[b], gsems[b])

        def cbody(i, carry, idx=idx):
            j0 = i * 4
            for b in range(4):
                j = j0 + b
                pltpu.make_async_copy(
                    xvr.at[idx.at[j]], bufs[b], gsems[b]).wait()
                pltpu.async_copy(bufs[b], shared.at[dst_v.at[j]], ssems[b],
                                 add=True)

                @pl.when(j + 4 < _EPT)
                def _():
                    pltpu.make_async_copy(
                        bufs[b], shared.at[dst_v.at[j]], ssems[b]).wait()
                    pltpu.async_copy(xvr.at[idx.at[j + 4]], bufs[b], gsems[b])
            return carry

        lax.fori_loop(0, _EPT // 4, cbody, 0)
        for b in range(4):
            pltpu.make_async_copy(
                bufs[b], shared.at[dst_v.at[_EPT - 4 + b]], ssems[b]).wait()
        plsc.subcore_barrier()

        def _mk_out(o):
            def go():
                pltpu.sync_copy(shared.at[pl.ds(s * _RPT, _RPT)],
                                o.at[pl.ds(s * _RPT, _RPT)])
            return go

        pl.when(c == 0)(_mk_out(outs[2 * p]))
        pl.when(c == 1)(_mk_out(outs[2 * p + 1]))
        plsc.subcore_barrier()

    pltpu.sync_copy(deg_v, degpr.at[wid])


def _sc_aggregate(xv, src2, dst2, zrows, zdeg):
    f32 = jnp.float32
    out_type = [jax.ShapeDtypeStruct((_BN, _Q), f32) for _ in range(4)]
    out_type.append(jax.ShapeDtypeStruct((_NC * _NS, _BN // 128, 128), f32))
    kern = pl.kernel(
        _sc_body,
        out_type=out_type,
        mesh=plsc.VectorSubcoreMesh(core_axis_name="c", subcore_axis_name="s"),
        compiler_params=pltpu.CompilerParams(needs_layout_passes=False,
                                              use_tc_tiling_on_sc=False),
        scratch_types=[
            pltpu.VMEM((_EPT, _CH), jnp.int32),
            pltpu.VMEM((_EPT, _CH), jnp.int32),
            pltpu.VMEM((_EPT, _CH), jnp.int32),
            pltpu.VMEM((_EPT, _CH), jnp.int32),
            pltpu.VMEM((_CH, _Q), f32),
            pltpu.VMEM((_CH, _Q), f32),
            pltpu.VMEM((_CH, _Q), f32),
            pltpu.VMEM((_CH, _Q), f32),
            pltpu.VMEM((_BN // 128, 128), f32),
            pltpu.VMEM_SHARED((_BN, _Q), f32),
            pltpu.SemaphoreType.DMA,
            pltpu.SemaphoreType.DMA,
            pltpu.SemaphoreType.DMA,
            pltpu.SemaphoreType.DMA,
            pltpu.SemaphoreType.DMA,
            pltpu.SemaphoreType.DMA,
            pltpu.SemaphoreType.DMA,
            pltpu.SemaphoreType.DMA,
        ],
    )
    return kern(xv, src2, dst2, zrows, zdeg)


def _softmax_mix_log(logits, valid, exp_temp, rand_coef):
    neg = jnp.float32(-1e9)
    masked = jnp.where(valid > 0, logits / exp_temp, neg)
    m = jnp.max(masked, axis=-1, keepdims=True)
    e = jnp.exp(masked - m)
    p = e / jnp.sum(e, axis=-1, keepdims=True)
    uni = valid / jnp.clip(jnp.sum(valid, axis=-1, keepdims=True), 1.0, None)
    probs = (1.0 - rand_coef) * p + rand_coef * uni
    return jnp.log(probs + 1e-20)


def _tc_main_body(xr, a0r, a1r, a2r, a3r, dgr, mr, wgr, bgr, w1dr, b1dr,
                  w2dr, w1br, b1br, w2br, sclr, hr, lpdr, lbwr):
    exp_temp = sclr[0, 0]
    rand_coef = sclr[0, 1]
    b_dep2 = sclr[0, 2]
    b_bw2 = sclr[0, 3]
    b16 = lambda a: a.astype(jnp.bfloat16)
    f = jnp.float32
    deg = jnp.clip(0.5 * jnp.sum(dgr[...], axis=0), 1.0, None)
    a = jnp.concatenate([a0r[...], a1r[...], a2r[...], a3r[...]], axis=1)
    x = xr[...] + a / deg[:, None]
    h = jnp.maximum(jnp.dot(b16(x), b16(wgr[...]), preferred_element_type=f)
                    + bgr[...], 0.0)
    hr[...] = h
    hd = jnp.maximum(jnp.dot(b16(h), b16(w1dr[...]), preferred_element_type=f)
                     + b1dr[...], 0.0)
    ld = jnp.sum(b16(hd).astype(f) * b16(w2dr[...]).astype(f),
                 axis=1).reshape(4, _N) + b_dep2
    hb = jnp.maximum(jnp.dot(b16(h), b16(w1br[...]), preferred_element_type=f)
                     + b1br[...], 0.0)
    lb = jnp.sum(b16(hb).astype(f) * b16(w2br[...]).astype(f),
                 axis=1).reshape(4, _N) + b_bw2
    dep_mask = jnp.max(mr[...], axis=1)
    lpdr[...] = _softmax_mix_log(ld, dep_mask, exp_temp, rand_coef)[None]
    col = lax.broadcasted_iota(jnp.int32, (4, _N), 1)
    bmask = (dep_mask <= 0.0) & (col > 0)
    lbwr[...] = jnp.where(bmask, lb, jnp.float32(-1e9))[None]


def _tc_main(x, a0, a1, a2, a3, degp, mask_f, W_gcn, b_gcn, W_dep1, b_dep1,
             w_dep2, W_bw1, b_bw1, w_bw2, scal):
    f32 = jnp.float32
    grid = _BN // 512
    return pl.pallas_call(
        _tc_main_body,
        grid=(grid,),
        in_specs=[
            pl.BlockSpec((512, _D), lambda i: (i, 0)),
            pl.BlockSpec((512, _Q), lambda i: (i, 0)),
            pl.BlockSpec((512, _Q), lambda i: (i, 0)),
            pl.BlockSpec((512, _Q), lambda i: (i, 0)),
            pl.BlockSpec((512, _Q), lambda i: (i, 0)),
            pl.BlockSpec((_NC * _NS, 512), lambda i: (0, i)),
            pl.BlockSpec((4, _N, _N), lambda i: (i, 0, 0)),
            pl.BlockSpec((_D, _D), lambda i: (0, 0)),
            pl.BlockSpec((1, _D), lambda i: (0, 0)),
            pl.BlockSpec((_D, 512), lambda i: (0, 0)),
            pl.BlockSpec((1, 512), lambda i: (0, 0)),
            pl.BlockSpec((1, 512), lambda i: (0, 0)),
            pl.BlockSpec((_D, 512), lambda i: (0, 0)),
            pl.BlockSpec((1, 512), lambda i: (0, 0)),
            pl.BlockSpec((1, 512), lambda i: (0, 0)),
            pl.BlockSpec(memory_space=pltpu.SMEM),
        ],
        out_specs=[
            pl.BlockSpec((512, _D), lambda i: (i, 0)),
            pl.BlockSpec((1, 4, _N), lambda i: (i, 0, 0)),
            pl.BlockSpec((1, 4, _N), lambda i: (i, 0, 0)),
        ],
        out_shape=[
            jax.ShapeDtypeStruct((_BN, _D), f32),
            jax.ShapeDtypeStruct((_B // 4, 4, _N), f32),
            jax.ShapeDtypeStruct((_B // 4, 4, _N), f32),
        ],
    )(x, a0, a1, a2, a3, degp, mask_f, W_gcn, b_gcn, W_dep1, b_dep1, w_dep2,
      W_bw1, b_bw1, w_bw2, scal)


def _tc_head_body(hr, mr, wr, ur, vr, lpdr, depr, sclr, lphr, lpfdr):
    b = pl.program_id(0)
    exp_temp = sclr[0, 0]
    rand_coef = sclr[0, 1]
    s_bi = sclr[0, 2]
    b16 = lambda a: a.astype(jnp.bfloat16)
    f = jnp.float32
    d = depr[b]
    hblk = hr[0]
    oh = (lax.broadcasted_iota(jnp.int32, (1, _N), 1) == d).astype(f)
    hd = jnp.sum(hblk * oh.reshape(_N, 1), axis=0, keepdims=True)
    t1a = jnp.dot(b16(hblk), b16(wr[...]), preferred_element_type=f)         + b16(ur[...]).astype(f)
    t1b = jnp.sum(b16(hblk).astype(f) * b16(vr[...]).astype(f), axis=1,
                  keepdims=True) + b16(jnp.float32(s_bi)).astype(f)
    lh = (jnp.sum(t1a * hd, axis=1, keepdims=True) + t1b).reshape(1, _N)
    hm = jnp.sum(mr[0] * oh, axis=1, keepdims=True).reshape(1, _N)
    lphr[...] = _softmax_mix_log(lh, hm, exp_temp, rand_coef)[None]
    lpfdr[...] = jnp.sum(lpdr[0] * oh, axis=-1, keepdims=True)[None]


def _tc_head(h3, mask_f, Wc, u_bi, w_bi, logp_dep, dep_ids, scal):
    f32 = jnp.float32
    return pl.pallas_call(
        _tc_head_body,
        grid=(_B,),
        in_specs=[
            pl.BlockSpec((1, _N, _D), lambda b: (b, 0, 0)),
            pl.BlockSpec((1, _N, _N), lambda b: (b, 0, 0)),
            pl.BlockSpec((_D, _D), lambda b: (0, 0)),
            pl.BlockSpec((1, _D), lambda b: (0, 0)),
            pl.BlockSpec((1, _D), lambda b: (0, 0)),
            pl.BlockSpec((1, 1, _N), lambda b: (b, 0, 0)),
            pl.BlockSpec(memory_space=pltpu.SMEM),
            pl.BlockSpec(memory_space=pltpu.SMEM),
        ],
        out_specs=[
            pl.BlockSpec((1, 1, _N), lambda b: (b, 0, 0)),
            pl.BlockSpec((1, 1, 1), lambda b: (b, 0, 0)),
        ],
        out_shape=[
            jax.ShapeDtypeStruct((_B, 1, _N), f32),
            jax.ShapeDtypeStruct((_B, 1, 1), f32),
        ],
    )(h3, mask_f, Wc, u_bi, w_bi, logp_dep.reshape(_B, 1, _N), dep_ids, scal)


def kernel(node_feats, edge_index, mask, exp_temp, rand_coef, W_gcn, b_gcn,
           W_dep1, b_dep1, W_dep2, b_dep2, W_bi, W_bw1, b_bw1, W_bw2, b_bw2):
    f32 = jnp.float32
    x = node_feats.astype(f32)
    src2 = edge_index[0].reshape(_E // _CH, _CH).astype(jnp.int32)
    dst2 = edge_index[1].reshape(_E // _CH, _CH).astype(jnp.int32)
    mask_f = mask.astype(f32)

    zrows = jnp.zeros((_BN, _Q), f32)
    zdeg = jnp.zeros((_BN // 128, 128), f32)
    o0, o1, o2, o3, degp = _sc_aggregate(x.reshape(_BN * 4, _Q), src2,
                                         dst2, zrows, zdeg)
    degp2 = degp.reshape(_NC * _NS, _BN)

    et = jnp.asarray(exp_temp, f32)
    rc = jnp.asarray(rand_coef, f32)
    scal1 = jnp.stack([et, rc, b_dep2.reshape(()).astype(f32),
                       b_bw2.reshape(()).astype(f32)]).reshape(1, 4)
    h, logp_dep, logits_bw = _tc_main(
        x, o0, o1, o2, o3, degp2, mask_f, W_gcn, b_gcn.reshape(1, _D),
        W_dep1, b_dep1.reshape(1, 512), W_dep2.reshape(1, 512),
        W_bw1, b_bw1.reshape(1, 512), W_bw2.reshape(1, 512), scal1)
    logp_dep = logp_dep.reshape(_B, _N)
    logits_bw = logits_bw.reshape(_B, _N)

    dep_ids = jax.random.categorical(jax.random.key(1), logp_dep,
                                     axis=-1)[:, None]

    Wc = W_bi[:_D, :_D]
    u_bi = W_bi[:_D, _D].reshape(1, _D)
    w_bi = W_bi[_D, :_D].reshape(1, _D)
    scal2 = jnp.stack([et, rc, W_bi[_D, _D].astype(f32)]).reshape(1, 3)
    logp_head, log_pF_dep = _tc_head(
        h.reshape(_B, _N, _D), mask_f, Wc, w_bi, u_bi, logp_dep,
        dep_ids[:, 0].astype(jnp.int32), scal2)
    logp_head = logp_head.reshape(_B, _N)
    log_pF_dep = log_pF_dep.reshape(_B, 1)

    head_ids = jax.random.categorical(jax.random.key(2), logp_head,
                                      axis=-1)[:, None]
    log_pF_head = jnp.take_along_axis(logp_head, head_ids, axis=-1)
    return (head_ids, dep_ids), (log_pF_head, log_pF_dep), logits_bw
